# baseline (speedup 1.0000x reference)
import functools

import jax
import jax.numpy as jnp
from jax import lax
from jax.experimental import pallas as pl
from jax.experimental.pallas import tpu as pltpu
from jax.experimental.pallas import tpu_sc as plsc

DIM = 1024
NUM_EXPERTS = 8
N_ROUTED = 7
HIDDEN = 4096
ALPHA = 2.0
TOKENS = 64
HB = 2048
NH = HIDDEN // HB
PADE = 16
NEG = -1e30


def _dotT(a, b):
    return jax.lax.dot_general(a, b, (((1,), (1,)), ((), ())),
                               preferred_element_type=jnp.float32)


def _swiglu_part(x, wu, wg, wd):
    up = _dotT(x, wu)
    g = _dotT(x, wg)
    act = up * (g * jax.nn.sigmoid(g))
    return _dotT(act, wd)


_SC_MESH = plsc.VectorSubcoreMesh(core_axis_name="c", subcore_axis_name="s")

NK = DIM // 16


@functools.partial(
    pl.kernel,
    mesh=_SC_MESH,
    out_type=jax.ShapeDtypeStruct((TOKENS, PADE), jnp.float32),
    scratch_types=[
        pltpu.VMEM((2, DIM), jnp.float32),
        pltpu.VMEM((NUM_EXPERTS, DIM), jnp.float32),
        pltpu.VMEM((2, PADE), jnp.float32),
    ],
)
def _sc_route(x_hbm, wr_hbm, scale_hbm, xbuf, wrbuf, sbuf):
    wid = lax.axis_index("s") * 2 + lax.axis_index("c")
    base = wid * 2
    pltpu.sync_copy(x_hbm.at[pl.ds(base, 2)], xbuf)
    pltpu.sync_copy(wr_hbm, wrbuf)
    idx = lax.iota(jnp.int32, 16)

    def _g(u, i):
        return lax.gather(
            u, i[:, None],
            lax.GatherDimensionNumbers(offset_dims=(),
                                       collapsed_slice_dims=(0,),
                                       start_index_map=(0,)),
            slice_sizes=(1,),
            mode=lax.GatherScatterMode.PROMISE_IN_BOUNDS)

    def _bfly(u, op):
        for sh in (8, 4, 2, 1):
            u = op(u, _g(u, idx ^ sh))
        return u

    for it in range(2):
        v = jnp.full((16,), -1e30, jnp.float32)
        for e in range(NUM_EXPERTS):
            acc = jnp.zeros((16,), jnp.float32)
            for k in range(NK):
                acc = acc + (xbuf[it, pl.ds(16 * k, 16)] *
                             wrbuf[e, pl.ds(16 * k, 16)])
            dot = _bfly(acc, jnp.add)
            v = jnp.where(idx == e, dot, v)
        m = _bfly(v, jnp.maximum)
        e_ = jnp.exp(v - m)
        p = _bfly(e_, jnp.maximum) / _bfly(e_, jnp.add)
        cand = jnp.where(v == m, idx, idx * 0 + 16)
        amin = _bfly(cand, jnp.minimum)
        sbuf[it] = jnp.where(idx == amin, ALPHA * p, jnp.zeros_like(v))
    pltpu.sync_copy(sbuf, scale_hbm.at[pl.ds(base, 2)])


def _shared_body(x_ref, wu_ref, wg_ref, wd_ref, out_ref):
    h = pl.program_id(0)

    @pl.when(h == 0)
    def _init():
        out_ref[...] = jnp.zeros_like(out_ref)

    out_ref[...] += _swiglu_part(x_ref[...], wu_ref[...], wg_ref[...],
                                 wd_ref[...])


def _routed_body(x_ref, wu_ref, wg_ref, wd_ref, out_ref):
    h = pl.program_id(1)

    @pl.when(h == 0)
    def _init():
        out_ref[...] = jnp.zeros_like(out_ref)

    out_ref[0] += _swiglu_part(x_ref[...], wu_ref[0], wg_ref[0], wd_ref[0])


def _combine_body(scale_ref, shared_ref, outs_ref, out_ref):
    acc = shared_ref[...]
    for j in range(N_ROUTED):
        acc = acc + outs_ref[j] * scale_ref[:, j][:, None]
    out_ref[...] = acc


@jax.jit
def kernel(x, Wr, Wu, Wg, Wd, Wu_s, Wg_s, Wd_s):
    scale = _sc_route(x, Wr)

    shared = pl.pallas_call(
        _shared_body,
        grid=(NH,),
        in_specs=[
            pl.BlockSpec((TOKENS, DIM), lambda h: (0, 0)),
            pl.BlockSpec((HB, DIM), lambda h: (h, 0)),
            pl.BlockSpec((HB, DIM), lambda h: (h, 0)),
            pl.BlockSpec((DIM, HB), lambda h: (0, h)),
        ],
        out_specs=pl.BlockSpec((TOKENS, DIM), lambda h: (0, 0)),
        out_shape=jax.ShapeDtypeStruct((TOKENS, DIM), jnp.float32),
        compiler_params=pltpu.CompilerParams(
            dimension_semantics=("arbitrary",),
        ),
    )(x, Wu_s, Wg_s, Wd_s)

    outs = pl.pallas_call(
        _routed_body,
        grid=(N_ROUTED, NH),
        in_specs=[
            pl.BlockSpec((TOKENS, DIM), lambda j, h: (0, 0)),
            pl.BlockSpec((1, HB, DIM), lambda j, h: (j, h, 0)),
            pl.BlockSpec((1, HB, DIM), lambda j, h: (j, h, 0)),
            pl.BlockSpec((1, DIM, HB), lambda j, h: (j, 0, h)),
        ],
        out_specs=pl.BlockSpec((1, TOKENS, DIM), lambda j, h: (j, 0, 0)),
        out_shape=jax.ShapeDtypeStruct((N_ROUTED, TOKENS, DIM), jnp.float32),
        compiler_params=pltpu.CompilerParams(
            dimension_semantics=("arbitrary", "arbitrary"),
        ),
    )(x, Wu, Wg, Wd)

    out = pl.pallas_call(
        _combine_body,
        in_specs=[
            pl.BlockSpec((TOKENS, PADE), lambda: (0, 0)),
            pl.BlockSpec((TOKENS, DIM), lambda: (0, 0)),
            pl.BlockSpec((N_ROUTED, TOKENS, DIM), lambda: (0, 0, 0)),
        ],
        out_specs=pl.BlockSpec((TOKENS, DIM), lambda: (0, 0)),
        out_shape=jax.ShapeDtypeStruct((TOKENS, DIM), jnp.float32),
    )(scale, shared, outs)
    return out


# speedup vs baseline: 1.0466x; 1.0466x over previous
import functools

import jax
import jax.numpy as jnp
from jax import lax
from jax.experimental import pallas as pl
from jax.experimental.pallas import tpu as pltpu
from jax.experimental.pallas import tpu_sc as plsc

DIM = 1024
NUM_EXPERTS = 8
N_ROUTED = 7
HIDDEN = 4096
ALPHA = 2.0
TOKENS = 64
HB = 2048
NH = HIDDEN // HB
PADE = 16
NEG = -1e30


def _dotT(a, b):
    return jax.lax.dot_general(a, b, (((1,), (1,)), ((), ())),
                               preferred_element_type=jnp.float32)


def _swiglu_part(x, wu, wg, wd):
    up = _dotT(x, wu)
    g = _dotT(x, wg)
    act = up * (g * jax.nn.sigmoid(g))
    return _dotT(act, wd)


_SC_MESH = plsc.VectorSubcoreMesh(core_axis_name="c", subcore_axis_name="s")


@functools.partial(
    pl.kernel,
    mesh=_SC_MESH,
    out_type=jax.ShapeDtypeStruct((TOKENS, PADE), jnp.float32),
    scratch_types=[
        pltpu.VMEM((2, PADE), jnp.float32),
        pltpu.VMEM((2, PADE), jnp.float32),
    ],
)
def _sc_route(logits_hbm, scale_hbm, lbuf, sbuf):
    wid = lax.axis_index("s") * 2 + lax.axis_index("c")
    base = wid * 2
    pltpu.sync_copy(logits_hbm.at[pl.ds(base, 2)], lbuf)
    idx = lax.iota(jnp.int32, 16)

    def _g(u, i):
        return lax.gather(
            u, i[:, None],
            lax.GatherDimensionNumbers(offset_dims=(),
                                       collapsed_slice_dims=(0,),
                                       start_index_map=(0,)),
            slice_sizes=(1,),
            mode=lax.GatherScatterMode.PROMISE_IN_BOUNDS)

    def _bfly(u, op):
        for sh in (8, 4, 2, 1):
            u = op(u, _g(u, idx ^ sh))
        return u

    for it in range(2):
        v = lbuf[it]
        m = _bfly(v, jnp.maximum)
        e = jnp.exp(v - m)
        p = _bfly(e, jnp.maximum) / _bfly(e, jnp.add)
        cand = jnp.where(v == m, idx, idx * 0 + 16)
        amin = _bfly(cand, jnp.minimum)
        sbuf[it] = jnp.where(idx == amin, ALPHA * p, jnp.zeros_like(v))
    pltpu.sync_copy(sbuf, scale_hbm.at[pl.ds(base, 2)])


def _shared_body(x_ref, wr_ref, wu_ref, wg_ref, wd_ref, out_ref, lg_ref):
    h = pl.program_id(0)

    @pl.when(h == 0)
    def _init():
        out_ref[...] = jnp.zeros_like(out_ref)
        lg = _dotT(x_ref[...], wr_ref[...])
        pad = jnp.full((TOKENS, PADE - NUM_EXPERTS), NEG, jnp.float32)
        lg_ref[...] = jnp.concatenate([lg, pad], axis=1)

    out_ref[...] += _swiglu_part(x_ref[...], wu_ref[...], wg_ref[...],
                                 wd_ref[...])


def _routed_body(x_ref, scale_ref, shared_ref, wu_ref, wg_ref, wd_ref,
                 out_ref):
    j = pl.program_id(0)
    h = pl.program_id(1)

    @pl.when((j == 0) & (h == 0))
    def _init():
        out_ref[...] = shared_ref[...]

    cols = jax.lax.broadcasted_iota(jnp.int32, (TOKENS, PADE), 1)
    s = jnp.sum(jnp.where(cols == j, scale_ref[...], 0.0), axis=1,
                keepdims=True)
    out_ref[...] += _swiglu_part(x_ref[...], wu_ref[0], wg_ref[0],
                                 wd_ref[0]) * s


@jax.jit
def kernel(x, Wr, Wu, Wg, Wd, Wu_s, Wg_s, Wd_s):

    shared, logits = pl.pallas_call(
        _shared_body,
        grid=(NH,),
        in_specs=[
            pl.BlockSpec((TOKENS, DIM), lambda h: (0, 0)),
            pl.BlockSpec((NUM_EXPERTS, DIM), lambda h: (0, 0)),
            pl.BlockSpec((HB, DIM), lambda h: (h, 0)),
            pl.BlockSpec((HB, DIM), lambda h: (h, 0)),
            pl.BlockSpec((DIM, HB), lambda h: (0, h)),
        ],
        out_specs=[
            pl.BlockSpec((TOKENS, DIM), lambda h: (0, 0)),
            pl.BlockSpec((TOKENS, PADE), lambda h: (0, 0)),
        ],
        out_shape=[
            jax.ShapeDtypeStruct((TOKENS, DIM), jnp.float32),
            jax.ShapeDtypeStruct((TOKENS, PADE), jnp.float32),
        ],
        compiler_params=pltpu.CompilerParams(
            dimension_semantics=("arbitrary",),
        ),
    )(x, Wr, Wu_s, Wg_s, Wd_s)

    scale = _sc_route(logits)

    out = pl.pallas_call(
        _routed_body,
        grid=(N_ROUTED, NH),
        in_specs=[
            pl.BlockSpec((TOKENS, DIM), lambda j, h: (0, 0)),
            pl.BlockSpec((TOKENS, PADE), lambda j, h: (0, 0)),
            pl.BlockSpec((TOKENS, DIM), lambda j, h: (0, 0)),
            pl.BlockSpec((1, HB, DIM), lambda j, h: (j, h, 0)),
            pl.BlockSpec((1, HB, DIM), lambda j, h: (j, h, 0)),
            pl.BlockSpec((1, DIM, HB), lambda j, h: (j, 0, h)),
        ],
        out_specs=pl.BlockSpec((TOKENS, DIM), lambda j, h: (0, 0)),
        out_shape=jax.ShapeDtypeStruct((TOKENS, DIM), jnp.float32),
        compiler_params=pltpu.CompilerParams(
            dimension_semantics=("arbitrary", "arbitrary"),
        ),
    )(x, scale, shared, Wu, Wg, Wd)
    return out
